# baseline (speedup 1.0000x reference)
import functools

import jax
import jax.numpy as jnp
from jax import lax
from jax.experimental import pallas as pl
from jax.experimental.pallas import tpu as pltpu
from jax.experimental.pallas import tpu_sc as plsc

_PH, _PW = 7, 7
_SCALE = 1.0
_NEG = float("-inf")
_L = 16


def _bin_bounds(start, end, nbins, dim):
    length = jnp.maximum(end - start + 1.0, 1.0)
    bsz = length / nbins
    p = jnp.arange(nbins, dtype=jnp.float32)
    lo = jnp.clip(jnp.floor(p[None, :] * bsz[:, None]) + start[:, None], 0.0, float(dim))
    hi = jnp.clip(jnp.ceil((p[None, :] + 1.0) * bsz[:, None]) + start[:, None], 0.0, float(dim))
    return lo.astype(jnp.int32), hi.astype(jnp.int32)


_KTC = 400


def _tc_body(meta_ref, x_ref, o_ref, *, H, W, C, max_span):
    i = pl.program_id(0)
    hs, he, ws, we = [], [], [], []
    for j in range(_PH):
        word = meta_ref[i, 2 + j]
        hs.append(word & 0xFF)
        he.append((word >> 8) & 0xFF)
        ws.append((word >> 16) & 0xFF)
        we.append((word >> 24) & 0xFF)

    wcoord = lax.broadcasted_iota(jnp.int32, (W, 1), 0)

    v1 = []
    for ph in range(_PH):
        base = jnp.minimum(hs[ph], H - max_span)
        rows = x_ref[0, pl.ds(base, max_span)]
        acc = jnp.full((W, C), _NEG, jnp.float32)
        for d in range(max_span):
            take = (base + d >= hs[ph]) & (base + d < he[ph])
            acc = jnp.where(take, jnp.maximum(acc, rows[d]), acc)
        v1.append(acc)

    out_rows = []
    for ph in range(_PH):
        h_empty = he[ph] <= hs[ph]
        for pw in range(_PW):
            mask = (wcoord >= ws[pw]) & (wcoord < we[pw])
            val = jnp.max(jnp.where(mask, v1[ph], _NEG), axis=0, keepdims=True)
            empty = h_empty | (we[pw] <= ws[pw])
            out_rows.append(jnp.where(empty, 0.0, val))
    o_ref[0] = jnp.concatenate(out_rows, axis=0)


_CPR = 4
_NCH = 8


def _sc_body(meta_hbm, xrows_hbm, out_hbm, meta_v, idx_a, idx_b, rows_a,
             rows_b, v1_v, sem_a, sem_b, *, M, NPR, NW, NC, H, W, C, NUNITS):
    wid = lax.axis_index("s") * NC + lax.axis_index("c")
    pltpu.sync_copy(meta_hbm.at[wid], meta_v)
    iota16 = lax.iota(jnp.int32, _L)
    neg16 = jnp.full((_L,), _NEG, jnp.float32)
    zero16 = jnp.zeros((_L,), jnp.float32)
    NG = C // _L
    NB = _PH * _PW * C // _L

    def roi_body(k, _):
        r = wid * NPR + k

        @pl.when(r < M)
        def _process():
            va = meta_v[k, pl.ds(0, _L)]
            vb = meta_v[k, pl.ds(_L, _L)]

            def g(j):
                return va[j] if j < _L else vb[j - _L]

            row0 = g(0)
            nrows = g(1)
            hs0 = g(2)
            ws = [g(3 + p) for p in range(_PW)]
            we = [g(10 + p) for p in range(_PW)]
            hsl = [g(17 + q) for q in range(_PH)]
            hel = [g(24 + q) for q in range(_PH)]

            def clr(i, _):
                for u in range(8):
                    v1_v[pl.ds((i * 8 + u) * _L, _L)] = neg16
                return 0
            lax.fori_loop(0, NB // 8, clr, 0)

            def start_gather(c, idx_ref, buf_ref, sem):
                units = jnp.minimum(4 * (row0 + c * _CPR) + iota16, NUNITS - 1)
                idx_ref[...] = units
                pltpu.make_async_copy(xrows_hbm.at[idx_ref], buf_ref,
                                      sem).start()

            def process(c, buf_ref):
                jmax = jnp.minimum(nrows - c * _CPR, _CPR)

                def row_body(j, _):
                    h = hs0 + c * _CPR + j
                    qlo = jnp.int32(0)
                    qhi = jnp.int32(0)
                    for q in range(_PH):
                        qlo += (hel[q] <= h).astype(jnp.int32)
                        qhi += (hsl[q] <= h).astype(jnp.int32)
                    for p in range(_PW):
                        def w_body(w, accs):
                            u = j * 4 + (w >> 3)
                            off = (w & 7) * C
                            return tuple(
                                jnp.maximum(accs[cg],
                                            buf_ref[u, pl.ds(off + cg * _L, _L)])
                                for cg in range(NG))
                        accs = lax.fori_loop(ws[p], we[p], w_body,
                                             (neg16,) * NG)

                        def fold_body(q, _, p=p, accs=accs):
                            for cg in range(NG):
                                sl = pl.ds((q * _PW + p) * C + cg * _L, _L)
                                v1_v[sl] = jnp.maximum(v1_v[sl], accs[cg])
                            return 0
                        lax.fori_loop(qlo, qhi, fold_body, 0)
                    return 0
                lax.fori_loop(0, jmax, row_body, 0)

            start_gather(jnp.int32(0), idx_a, rows_a, sem_a)

            def pair(t, _):
                c0 = 2 * t
                c1 = 2 * t + 1

                @pl.when(c1 * _CPR < nrows)
                def _pre_b():
                    start_gather(c1, idx_b, rows_b, sem_b)

                @pl.when(c0 * _CPR < nrows)
                def _run_a():
                    pltpu.make_async_copy(xrows_hbm.at[idx_a], rows_a,
                                          sem_a).wait()
                    process(c0, rows_a)

                @pl.when((c1 + 1) * _CPR < nrows)
                def _pre_a():
                    start_gather(c1 + 1, idx_a, rows_a, sem_a)

                @pl.when(c1 * _CPR < nrows)
                def _run_b():
                    pltpu.make_async_copy(xrows_hbm.at[idx_b], rows_b,
                                          sem_b).wait()
                    process(c1, rows_b)
                return 0
            lax.fori_loop(0, _NCH // 2, pair, 0)

            def fin(i, _):
                for u in range(8):
                    sl = pl.ds((i * 8 + u) * _L, _L)
                    v = v1_v[sl]
                    v1_v[sl] = jnp.where(v == neg16, zero16, v)
                return 0
            lax.fori_loop(0, NB // 8, fin, 0)
            pltpu.sync_copy(v1_v, out_hbm.at[r])
        return 0

    lax.fori_loop(0, NPR, roi_body, 0)


def kernel(input, rois):
    N, C, H, W = input.shape
    M = rois.shape[0]
    info = plsc.get_sparse_core_info()
    NC, NS = info.num_cores, info.num_subcores
    NW = NC * NS
    KTC = min(_KTC, M)
    MSC = M - KTC
    NPR = -(-MSC // NW) if MSC else 1
    MP = NW * NPR
    NUNITS = 4 * N * H
    max_span = -(-W // _PW) + 2

    xt = jnp.transpose(input, (0, 2, 3, 1))
    xrows = xt.reshape(NUNITS, W * C // 4)

    b = rois[:, 0].astype(jnp.int32)
    start_w = jnp.round(rois[:, 1] * _SCALE)
    start_h = jnp.round(rois[:, 2] * _SCALE)
    end_w = jnp.round(rois[:, 3] * _SCALE)
    end_h = jnp.round(rois[:, 4] * _SCALE)
    hs, he = _bin_bounds(start_h, end_h, _PH, H)
    ws, we = _bin_bounds(start_w, end_w, _PW, W)

    packed = hs[:KTC] | (he[:KTC] << 8) | (ws[:KTC] << 16) | (we[:KTC] << 24)
    b_tc = b[:KTC]
    order = jnp.argsort(b_tc)
    meta_tc = jnp.concatenate(
        [b_tc[order][:, None], order[:, None].astype(jnp.int32),
         packed[order]], axis=1)
    grid_spec = pltpu.PrefetchScalarGridSpec(
        num_scalar_prefetch=1,
        grid=(KTC,),
        in_specs=[
            pl.BlockSpec((1, H, W, C), lambda i, m: (m[i, 0], 0, 0, 0)),
        ],
        out_specs=pl.BlockSpec((1, _PH * _PW, C), lambda i, m: (m[i, 1], 0, 0)),
    )
    out_tc = pl.pallas_call(
        functools.partial(_tc_body, H=H, W=W, C=C, max_span=max_span),
        grid_spec=grid_spec,
        out_shape=jax.ShapeDtypeStruct((KTC, _PH * _PW, C), jnp.float32),
    )(meta_tc, xt)

    row0 = b[KTC:] * H + hs[KTC:, 0]
    nrows = he[KTC:, -1] - hs[KTC:, 0]
    meta_sc = jnp.concatenate(
        [row0[:, None], nrows[:, None], hs[KTC:, 0][:, None],
         ws[KTC:], we[KTC:], hs[KTC:], he[KTC:],
         jnp.zeros((MSC, 1), jnp.int32)], axis=1)
    meta_sc = jnp.pad(meta_sc, ((0, MP - MSC), (0, 0))).reshape(NW, NPR, 32)

    mesh = plsc.VectorSubcoreMesh(core_axis_name="c", subcore_axis_name="s")
    body = functools.partial(_sc_body, M=MSC, NPR=NPR, NW=NW, NC=NC,
                             H=H, W=W, C=C, NUNITS=NUNITS)
    sc_call = pl.kernel(
        body,
        mesh=mesh,
        out_type=jax.ShapeDtypeStruct((MSC, _PH * _PW * C), jnp.float32),
        scratch_types=[
            pltpu.VMEM((NPR, 32), jnp.int32),
            pltpu.VMEM((_L,), jnp.int32),
            pltpu.VMEM((_L,), jnp.int32),
            pltpu.VMEM((_L, W * C // 4), jnp.float32),
            pltpu.VMEM((_L, W * C // 4), jnp.float32),
            pltpu.VMEM((_PH * _PW * C,), jnp.float32),
            pltpu.SemaphoreType.DMA,
            pltpu.SemaphoreType.DMA,
        ],
    )
    out_sc = sc_call(meta_sc, xrows)

    out = jnp.concatenate([out_tc, out_sc.reshape(MSC, _PH * _PW, C)], axis=0)
    return out.transpose(0, 2, 1).reshape(M, C, _PH, _PW)


# speedup vs baseline: 1.0014x; 1.0014x over previous
import functools

import jax
import jax.numpy as jnp
from jax import lax
from jax.experimental import pallas as pl
from jax.experimental.pallas import tpu as pltpu
from jax.experimental.pallas import tpu_sc as plsc

_PH, _PW = 7, 7
_SCALE = 1.0
_NEG = float("-inf")
_L = 16


def _bin_bounds(start, end, nbins, dim):
    length = jnp.maximum(end - start + 1.0, 1.0)
    bsz = length / nbins
    p = jnp.arange(nbins, dtype=jnp.float32)
    lo = jnp.clip(jnp.floor(p[None, :] * bsz[:, None]) + start[:, None], 0.0, float(dim))
    hi = jnp.clip(jnp.ceil((p[None, :] + 1.0) * bsz[:, None]) + start[:, None], 0.0, float(dim))
    return lo.astype(jnp.int32), hi.astype(jnp.int32)


_KTC = 400


def _tc_body(meta_ref, x_ref, o_ref, *, H, W, C, max_span):
    i = pl.program_id(0)
    hs, he, ws, we = [], [], [], []
    for j in range(_PH):
        word = meta_ref[i, 2 + j]
        hs.append(word & 0xFF)
        he.append((word >> 8) & 0xFF)
        ws.append((word >> 16) & 0xFF)
        we.append((word >> 24) & 0xFF)

    wcoord = lax.broadcasted_iota(jnp.int32, (W, 1), 0)

    v1 = []
    for ph in range(_PH):
        base = jnp.minimum(hs[ph], H - max_span)
        rows = x_ref[0, pl.ds(base, max_span)]
        acc = jnp.full((W, C), _NEG, jnp.float32)
        for d in range(max_span):
            take = (base + d >= hs[ph]) & (base + d < he[ph])
            acc = jnp.where(take, jnp.maximum(acc, rows[d]), acc)
        v1.append(acc)

    out_rows = []
    for ph in range(_PH):
        h_empty = he[ph] <= hs[ph]
        for pw in range(_PW):
            mask = (wcoord >= ws[pw]) & (wcoord < we[pw])
            val = jnp.max(jnp.where(mask, v1[ph], _NEG), axis=0, keepdims=True)
            empty = h_empty | (we[pw] <= ws[pw])
            out_rows.append(jnp.where(empty, 0.0, val))
    o_ref[0] = jnp.concatenate(out_rows, axis=0)


_CPR = 4
_NCH = 8


def _sc_body(meta_hbm, xrows_hbm, out_hbm, meta_v, idx_a, idx_b, rows_a,
             rows_b, v1_v, sem_a, sem_b, *, M, NPR, NW, NC, H, W, C, NUNITS):
    wid = lax.axis_index("s") * NC + lax.axis_index("c")
    pltpu.sync_copy(meta_hbm.at[wid], meta_v)
    iota16 = lax.iota(jnp.int32, _L)
    neg16 = jnp.full((_L,), _NEG, jnp.float32)
    zero16 = jnp.zeros((_L,), jnp.float32)
    NG = C // _L
    NB = _PH * _PW * C // _L

    def roi_body(k, _):
        r = wid * NPR + k

        @pl.when(r < M)
        def _process():
            va = meta_v[k, pl.ds(0, _L)]
            vb = meta_v[k, pl.ds(_L, _L)]

            def g(j):
                return va[j] if j < _L else vb[j - _L]

            row0 = g(0)
            nrows = g(1)
            hs0 = g(2)
            ws = [g(3 + p) for p in range(_PW)]
            we = [g(10 + p) for p in range(_PW)]
            hsl = [g(17 + q) for q in range(_PH)]
            hel = [g(24 + q) for q in range(_PH)]

            def clr(i, _):
                for u in range(8):
                    v1_v[pl.ds((i * 8 + u) * _L, _L)] = neg16
                return 0
            lax.fori_loop(0, NB // 8, clr, 0)

            def start_gather(c, idx_ref, buf_ref, sem):
                units = jnp.minimum(4 * (row0 + c * _CPR) + iota16, NUNITS - 1)
                idx_ref[...] = units
                pltpu.make_async_copy(xrows_hbm.at[idx_ref], buf_ref,
                                      sem).start()

            def process(c, buf_ref):
                jmax = jnp.minimum(nrows - c * _CPR, _CPR)

                def row_body(j, _):
                    h = hs0 + c * _CPR + j
                    qlo = jnp.int32(0)
                    qhi = jnp.int32(0)
                    for q in range(_PH):
                        qlo += (hel[q] <= h).astype(jnp.int32)
                        qhi += (hsl[q] <= h).astype(jnp.int32)
                    for p in range(_PW):
                        def w_body(w, accs):
                            u = j * 4 + (w >> 3)
                            off = (w & 7) * C
                            return tuple(
                                jnp.maximum(accs[cg],
                                            buf_ref[u, pl.ds(off + cg * _L, _L)])
                                for cg in range(NG))
                        accs = lax.fori_loop(ws[p], we[p], w_body,
                                             (neg16,) * NG)

                        def fold_body(q, _, p=p, accs=accs):
                            for cg in range(NG):
                                sl = pl.ds((q * _PW + p) * C + cg * _L, _L)
                                v1_v[sl] = jnp.maximum(v1_v[sl], accs[cg])
                            return 0
                        lax.fori_loop(qlo, qhi, fold_body, 0)
                    return 0
                lax.fori_loop(0, jmax, row_body, 0)

            start_gather(jnp.int32(0), idx_a, rows_a, sem_a)

            def pair(t, _):
                c0 = 2 * t
                c1 = 2 * t + 1

                @pl.when(c1 * _CPR < nrows)
                def _pre_b():
                    start_gather(c1, idx_b, rows_b, sem_b)

                @pl.when(c0 * _CPR < nrows)
                def _run_a():
                    pltpu.make_async_copy(xrows_hbm.at[idx_a], rows_a,
                                          sem_a).wait()
                    process(c0, rows_a)

                @pl.when((c1 + 1) * _CPR < nrows)
                def _pre_a():
                    start_gather(c1 + 1, idx_a, rows_a, sem_a)

                @pl.when(c1 * _CPR < nrows)
                def _run_b():
                    pltpu.make_async_copy(xrows_hbm.at[idx_b], rows_b,
                                          sem_b).wait()
                    process(c1, rows_b)
                return 0
            lax.fori_loop(0, _NCH // 2, pair, 0)

            def fin(i, _):
                for u in range(8):
                    sl = pl.ds((i * 8 + u) * _L, _L)
                    v = v1_v[sl]
                    v1_v[sl] = jnp.where(v == neg16, zero16, v)
                return 0
            lax.fori_loop(0, NB // 8, fin, 0)
            pltpu.sync_copy(v1_v, out_hbm.at[r])
        return 0

    lax.fori_loop(0, NPR, roi_body, 0)


def kernel(input, rois):
    N, C, H, W = input.shape
    M = rois.shape[0]
    info = plsc.get_sparse_core_info()
    NC, NS = info.num_cores, info.num_subcores
    NW = NC * NS
    KTC = min(_KTC, M)
    MSC = M - KTC
    NPR = -(-MSC // NW) if MSC else 1
    MP = NW * NPR
    NUNITS = 4 * N * H
    max_span = -(-W // _PW) + 2

    xt = jnp.transpose(input, (0, 2, 3, 1))
    xrows = xt.reshape(NUNITS, W * C // 4)

    b = rois[:, 0].astype(jnp.int32)
    start_w = jnp.round(rois[:, 1] * _SCALE)
    start_h = jnp.round(rois[:, 2] * _SCALE)
    end_w = jnp.round(rois[:, 3] * _SCALE)
    end_h = jnp.round(rois[:, 4] * _SCALE)
    hs, he = _bin_bounds(start_h, end_h, _PH, H)
    ws, we = _bin_bounds(start_w, end_w, _PW, W)

    row0 = b[KTC:] * H + hs[KTC:, 0]
    nrows = he[KTC:, -1] - hs[KTC:, 0]
    meta_sc = jnp.concatenate(
        [row0[:, None], nrows[:, None], hs[KTC:, 0][:, None],
         ws[KTC:], we[KTC:], hs[KTC:], he[KTC:],
         jnp.zeros((MSC, 1), jnp.int32)], axis=1)
    meta_sc = jnp.pad(meta_sc, ((0, MP - MSC), (0, 0))).reshape(NW, NPR, 32)

    mesh = plsc.VectorSubcoreMesh(core_axis_name="c", subcore_axis_name="s")
    body = functools.partial(_sc_body, M=MSC, NPR=NPR, NW=NW, NC=NC,
                             H=H, W=W, C=C, NUNITS=NUNITS)
    sc_call = pl.kernel(
        body,
        mesh=mesh,
        out_type=jax.ShapeDtypeStruct((MSC, _PH * _PW * C), jnp.float32),
        scratch_types=[
            pltpu.VMEM((NPR, 32), jnp.int32),
            pltpu.VMEM((_L,), jnp.int32),
            pltpu.VMEM((_L,), jnp.int32),
            pltpu.VMEM((_L, W * C // 4), jnp.float32),
            pltpu.VMEM((_L, W * C // 4), jnp.float32),
            pltpu.VMEM((_PH * _PW * C,), jnp.float32),
            pltpu.SemaphoreType.DMA,
            pltpu.SemaphoreType.DMA,
        ],
    )
    out_sc = sc_call(meta_sc, xrows)

    packed = hs[:KTC] | (he[:KTC] << 8) | (ws[:KTC] << 16) | (we[:KTC] << 24)
    b_tc = b[:KTC]
    order = jnp.argsort(b_tc)
    meta_tc = jnp.concatenate(
        [b_tc[order][:, None], order[:, None].astype(jnp.int32),
         packed[order]], axis=1)
    grid_spec = pltpu.PrefetchScalarGridSpec(
        num_scalar_prefetch=1,
        grid=(KTC,),
        in_specs=[
            pl.BlockSpec((1, H, W, C), lambda i, m: (m[i, 0], 0, 0, 0)),
        ],
        out_specs=pl.BlockSpec((1, _PH * _PW, C), lambda i, m: (m[i, 1], 0, 0)),
    )
    out_tc = pl.pallas_call(
        functools.partial(_tc_body, H=H, W=W, C=C, max_span=max_span),
        grid_spec=grid_spec,
        out_shape=jax.ShapeDtypeStruct((KTC, _PH * _PW, C), jnp.float32),
    )(meta_tc, xt)

    out = jnp.concatenate([out_tc, out_sc.reshape(MSC, _PH * _PW, C)], axis=0)
    return out.transpose(0, 2, 1).reshape(M, C, _PH, _PW)


# speedup vs baseline: 1.1546x; 1.1530x over previous
import functools

import jax
import jax.numpy as jnp
from jax import lax
from jax.experimental import pallas as pl
from jax.experimental.pallas import tpu as pltpu
from jax.experimental.pallas import tpu_sc as plsc

_PH, _PW = 7, 7
_SCALE = 1.0
_NEG = float("-inf")
_L = 16


def _bin_bounds(start, end, nbins, dim):
    length = jnp.maximum(end - start + 1.0, 1.0)
    bsz = length / nbins
    p = jnp.arange(nbins, dtype=jnp.float32)
    lo = jnp.clip(jnp.floor(p[None, :] * bsz[:, None]) + start[:, None], 0.0, float(dim))
    hi = jnp.clip(jnp.ceil((p[None, :] + 1.0) * bsz[:, None]) + start[:, None], 0.0, float(dim))
    return lo.astype(jnp.int32), hi.astype(jnp.int32)


_CPR = 4
_NCH = 8


def _sc_body(meta_hbm, xrows_hbm, out_hbm, meta_v, idx_a, idx_b, idx_c,
             rows_a, rows_b, rows_c, v1_v, o_v, sem_a, sem_b, sem_c,
             sem_o, *, M, NPR, NW, NC, H, W, C, NUNITS):
    wid = lax.axis_index("s") * NC + lax.axis_index("c")
    pltpu.sync_copy(meta_hbm.at[pl.ds(wid * NPR, NPR)], meta_v)
    iota16 = lax.iota(jnp.int32, _L)
    neg16 = jnp.full((_L,), _NEG, jnp.float32)
    zero16 = jnp.zeros((_L,), jnp.float32)
    NG = C // _L
    NB = _PH * _PW * C // _L

    def start_c0(kn):
        van = meta_v[kn, pl.ds(0, _L)]
        units = jnp.minimum(4 * van[0] + iota16, NUNITS - 1)
        idx_c[...] = units
        pltpu.make_async_copy(xrows_hbm.at[idx_c], rows_c, sem_c).start()

    @pl.when(wid * NPR < M)
    def _prime():
        start_c0(jnp.int32(0))

    def roi_body(k, _):
        r = wid * NPR + k

        @pl.when(r < M)
        def _process():
            va = meta_v[k, pl.ds(0, _L)]
            vb = meta_v[k, pl.ds(_L, _L)]

            def g(j):
                return va[j] if j < _L else vb[j - _L]

            row0 = g(0)
            nrows = g(1)
            hs0 = g(2)
            ws = [g(3 + p) for p in range(_PW)]
            we = [g(10 + p) for p in range(_PW)]
            hsl = [g(17 + q) for q in range(_PH)]
            hel = [g(24 + q) for q in range(_PH)]

            def clr(i, _):
                for u in range(8):
                    v1_v[pl.ds((i * 8 + u) * _L, _L)] = neg16
                return 0
            lax.fori_loop(0, NB // 8, clr, 0)

            def start_gather(c, idx_ref, buf_ref, sem):
                units = jnp.minimum(4 * (row0 + c * _CPR) + iota16, NUNITS - 1)
                idx_ref[...] = units
                pltpu.make_async_copy(xrows_hbm.at[idx_ref], buf_ref,
                                      sem).start()

            def process(c, buf_ref):
                jmax = jnp.minimum(nrows - c * _CPR, _CPR)

                def row_body(j, _):
                    h = hs0 + c * _CPR + j
                    qlo = jnp.int32(0)
                    qhi = jnp.int32(0)
                    for q in range(_PH):
                        qlo += (hel[q] <= h).astype(jnp.int32)
                        qhi += (hsl[q] <= h).astype(jnp.int32)
                    for p in range(_PW):
                        def w_body(w, accs):
                            u = j * 4 + (w >> 3)
                            off = (w & 7) * C
                            return tuple(
                                jnp.maximum(accs[cg],
                                            buf_ref[u, pl.ds(off + cg * _L, _L)])
                                for cg in range(NG))
                        accs = lax.fori_loop(ws[p], we[p], w_body,
                                             (neg16,) * NG)

                        def fold_body(q, _, p=p, accs=accs):
                            for cg in range(NG):
                                sl = pl.ds((q * _PW + p) * C + cg * _L, _L)
                                v1_v[sl] = jnp.maximum(v1_v[sl], accs[cg])
                            return 0
                        lax.fori_loop(qlo, qhi, fold_body, 0)
                    return 0
                lax.fori_loop(0, jmax, row_body, 0)

            @pl.when(_CPR < nrows)
            def _pre1():
                start_gather(jnp.int32(1), idx_a, rows_a, sem_a)

            pltpu.make_async_copy(xrows_hbm.at[idx_c], rows_c, sem_c).wait()
            process(jnp.int32(0), rows_c)

            @pl.when((r + 1 < M) & (k + 1 < NPR))
            def _pren():
                start_c0(k + 1)

            def pair(t2, _):
                c0 = 2 * t2 + 1
                c1 = 2 * t2 + 2

                @pl.when(c1 * _CPR < nrows)
                def _pre_b():
                    start_gather(c1, idx_b, rows_b, sem_b)

                @pl.when(c0 * _CPR < nrows)
                def _run_a():
                    pltpu.make_async_copy(xrows_hbm.at[idx_a], rows_a,
                                          sem_a).wait()
                    process(c0, rows_a)

                @pl.when((c1 + 1) * _CPR < nrows)
                def _pre_a():
                    start_gather(c1 + 1, idx_a, rows_a, sem_a)

                @pl.when(c1 * _CPR < nrows)
                def _run_b():
                    pltpu.make_async_copy(xrows_hbm.at[idx_b], rows_b,
                                          sem_b).wait()
                    process(c1, rows_b)
                return 0
            lax.fori_loop(0, _NCH // 2, pair, 0)

            @pl.when(k > 0)
            def _wprev():
                pltpu.make_async_copy(o_v, out_hbm.at[r - 1], sem_o).wait()

            def fin(i, _):
                for u in range(8):
                    sl = pl.ds((i * 8 + u) * _L, _L)
                    v = v1_v[sl]
                    o_v[sl] = jnp.where(v == neg16, zero16, v)
                return 0
            lax.fori_loop(0, NB // 8, fin, 0)
            pltpu.make_async_copy(o_v, out_hbm.at[r], sem_o).start()
        return 0

    lax.fori_loop(0, NPR, roi_body, 0)

    @pl.when(wid * NPR < M)
    def _drain_o():
        pltpu.make_async_copy(o_v, out_hbm.at[wid * NPR], sem_o).wait()


def kernel(input, rois):
    N, C, H, W = input.shape
    M = rois.shape[0]
    info = plsc.get_sparse_core_info()
    NC, NS = info.num_cores, info.num_subcores
    NW = NC * NS
    NPR = -(-M // NW)
    MP = NW * NPR
    NUNITS = 4 * N * H

    xrows = jnp.transpose(input, (0, 2, 3, 1)).reshape(NUNITS, W * C // 4)

    b = rois[:, 0].astype(jnp.int32)
    start_w = jnp.round(rois[:, 1] * _SCALE)
    start_h = jnp.round(rois[:, 2] * _SCALE)
    end_w = jnp.round(rois[:, 3] * _SCALE)
    end_h = jnp.round(rois[:, 4] * _SCALE)
    hs, he = _bin_bounds(start_h, end_h, _PH, H)
    ws, we = _bin_bounds(start_w, end_w, _PW, W)
    row0 = b * H + hs[:, 0]
    nrows = he[:, -1] - hs[:, 0]
    meta = jnp.concatenate(
        [row0[:, None], nrows[:, None], hs[:, 0][:, None],
         ws, we, hs, he, jnp.zeros((M, 1), jnp.int32)], axis=1)
    meta = jnp.pad(meta, ((0, MP - M), (0, 0)))

    mesh = plsc.VectorSubcoreMesh(core_axis_name="c", subcore_axis_name="s")
    body = functools.partial(_sc_body, M=M, NPR=NPR, NW=NW, NC=NC,
                             H=H, W=W, C=C, NUNITS=NUNITS)
    sc_call = pl.kernel(
        body,
        mesh=mesh,
        out_type=jax.ShapeDtypeStruct((M, _PH * _PW * C), jnp.float32),
        scratch_types=[
            pltpu.VMEM((NPR, 32), jnp.int32),
            pltpu.VMEM((_L,), jnp.int32),
            pltpu.VMEM((_L,), jnp.int32),
            pltpu.VMEM((_L,), jnp.int32),
            pltpu.VMEM((_L, W * C // 4), jnp.float32),
            pltpu.VMEM((_L, W * C // 4), jnp.float32),
            pltpu.VMEM((_L, W * C // 4), jnp.float32),
            pltpu.VMEM((_PH * _PW * C,), jnp.float32),
            pltpu.VMEM((_PH * _PW * C,), jnp.float32),
            pltpu.SemaphoreType.DMA,
            pltpu.SemaphoreType.DMA,
            pltpu.SemaphoreType.DMA,
            pltpu.SemaphoreType.DMA,
        ],
    )
    out = sc_call(meta, xrows)
    return out.reshape(M, _PH * _PW, C).transpose(0, 2, 1).reshape(M, C, _PH, _PW)
